# Initial kernel scaffold; baseline (speedup 1.0000x reference)
import jax
import jax.numpy as jnp
from jax.experimental import pallas as pl


def kernel(pos, h, adj, embedding):
    raise NotImplementedError("write your pallas kernel here")


# speedup vs baseline: 2.4709x; 2.4709x over previous
import functools
import jax
import jax.numpy as jnp
from jax import lax
from jax.experimental import pallas as pl
from jax.experimental.pallas import tpu as pltpu
from jax.experimental.pallas import tpu_sc as plsc

B, N = 1024, 128
D = 128
ROWS = B * N
NW = 32
ROWS_PER_W = ROWS // NW
C = 256
NCHUNK = ROWS_PER_W // C
IDX_ROWS = C // 128
GROUPS = C // 16

_mesh = plsc.VectorSubcoreMesh(core_axis_name="c", subcore_axis_name="s")


@functools.partial(
    pl.kernel,
    out_type=jax.ShapeDtypeStruct((ROWS, D), jnp.float32),
    mesh=_mesh,
    scratch_types=[
        pltpu.VMEM((2, IDX_ROWS, 128), jnp.int32),
        pltpu.VMEM((2, C, D), jnp.float32),
        pltpu.VMEM((2, 3 * C), jnp.float32),
        pltpu.SemaphoreType.DMA,
        pltpu.SemaphoreType.DMA,
        pltpu.SemaphoreType.DMA,
        pltpu.SemaphoreType.DMA,
    ],
    compiler_params=pltpu.CompilerParams(needs_layout_passes=False),
)
def _sc_encode(tbl_hbm, h_hbm, pos_hbm, out_hbm, idx_v, rows_v, pos_v, g0, g1, o0, o1):
    wid = lax.axis_index("s") * 2 + lax.axis_index("c")
    gsem = (g0, g1)
    osem = (o0, o1)

    lane = jax.lax.broadcasted_iota(jnp.int32, (16,), 0)
    rowoff = []
    coloff = []
    for k in range(3):
        t = lane + (16 * k)
        rowoff.append(t // 3)
        coloff.append(t - 3 * (t // 3))

    def gather_descs(g, p):
        base = wid * ROWS_PER_W + g * C
        return [
            pltpu.make_async_copy(
                tbl_hbm.at[idx_v.at[p, j]],
                rows_v.at[p, pl.ds(j * 128, 128)],
                gsem[p],
            )
            for j in range(IDX_ROWS)
        ]

    def out_desc(g, p):
        base = wid * ROWS_PER_W + g * C
        return pltpu.make_async_copy(
            rows_v.at[p], out_hbm.at[pl.ds(base, C)], osem[p]
        )

    def load_chunk(g, p):
        base = wid * ROWS_PER_W + g * C
        for j in range(IDX_ROWS):
            pltpu.sync_copy(h_hbm.at[pl.ds(base + j * 128, 128)], idx_v.at[p, j])
        pltpu.sync_copy(pos_hbm.at[pl.ds(base * 3, 3 * C)], pos_v.at[p])
        for d in gather_descs(g, p):
            d.start()

    def merge(p):
        def body(gr, carry):
            for k in range(3):
                vals = pos_v[p, pl.ds(gr * 48 + k * 16, 16)]
                plsc.store_scatter(
                    rows_v.at[p], [rowoff[k] + gr * 16, coloff[k]], vals
                )
            return carry

        lax.fori_loop(0, GROUPS, body, 0)

    load_chunk(0, 0)
    for g in range(NCHUNK):
        p = g % 2
        for d in gather_descs(g, p):
            d.wait()
        if g + 1 < NCHUNK:
            if g >= 1:
                out_desc(g - 1, 1 - p).wait()
            load_chunk(g + 1, 1 - p)
        merge(p)
        out_desc(g, p).start()
    out_desc(NCHUNK - 1, (NCHUNK - 1) % 2).wait()


def kernel(pos, h, adj, embedding):
    emb0 = embedding.at[0].set(0.0)
    tbl = jnp.concatenate(
        [jnp.zeros((emb0.shape[0], 3), jnp.float32), emb0], axis=1
    )
    out = _sc_encode(tbl, h.reshape(-1), pos.reshape(-1))
    return out.reshape(B, N, D)


# speedup vs baseline: 3.6881x; 1.4926x over previous
import functools
import jax
import jax.numpy as jnp
from jax import lax
from jax.experimental import pallas as pl
from jax.experimental.pallas import tpu as pltpu
from jax.experimental.pallas import tpu_sc as plsc

B, N = 1024, 128
D = 128
ROWS = B * N
NW = 32
ROWS_PER_W = ROWS // NW
C = 256
NCHUNK = ROWS_PER_W // C
IDX_ROWS = C // 128
GROUPS = C // 16

_mesh = plsc.VectorSubcoreMesh(core_axis_name="c", subcore_axis_name="s")


@functools.partial(
    pl.kernel,
    out_type=jax.ShapeDtypeStruct((ROWS, D), jnp.float32),
    mesh=_mesh,
    scratch_types=[
        pltpu.VMEM((2, IDX_ROWS, 128), jnp.int32),
        pltpu.VMEM((2, C, D), jnp.float32),
        pltpu.VMEM((2, 3 * C), jnp.float32),
        pltpu.VMEM_SHARED((119, 128), jnp.float32),
        pltpu.SemaphoreType.DMA,
        pltpu.SemaphoreType.DMA,
        pltpu.SemaphoreType.DMA,
        pltpu.SemaphoreType.DMA,
    ],
    compiler_params=pltpu.CompilerParams(needs_layout_passes=False),
)
def _sc_encode(
    tbl_hbm, h_hbm, pos_hbm, out_hbm, idx_v, rows_v, pos_v, tbl_sh, g0, g1, o0, o1
):
    wid = lax.axis_index("s") * 2 + lax.axis_index("c")
    gsem = (g0, g1)
    osem = (o0, o1)

    @pl.when(lax.axis_index("s") == 0)
    def _stage():
        pltpu.sync_copy(tbl_hbm, tbl_sh)

    plsc.subcore_barrier()

    lane = jax.lax.broadcasted_iota(jnp.int32, (16,), 0)
    rowoff = []
    coloff = []
    for k in range(3):
        t = lane + (16 * k)
        rowoff.append(t // 3)
        coloff.append(t - 3 * (t // 3))

    def gather_descs(g, p):
        base = wid * ROWS_PER_W + g * C
        return [
            pltpu.make_async_copy(
                tbl_sh.at[idx_v.at[p, j]],
                rows_v.at[p, pl.ds(j * 128, 128)],
                gsem[p],
            )
            for j in range(IDX_ROWS)
        ]

    def out_desc(g, p):
        base = wid * ROWS_PER_W + g * C
        return pltpu.make_async_copy(
            rows_v.at[p], out_hbm.at[pl.ds(base, C)], osem[p]
        )

    def load_chunk(g, p):
        base = wid * ROWS_PER_W + g * C
        for j in range(IDX_ROWS):
            pltpu.sync_copy(h_hbm.at[pl.ds(base + j * 128, 128)], idx_v.at[p, j])
        pltpu.sync_copy(pos_hbm.at[pl.ds(base * 3, 3 * C)], pos_v.at[p])
        for d in gather_descs(g, p):
            d.start()

    def merge(p):
        def body(gr, carry):
            for k in range(3):
                vals = pos_v[p, pl.ds(gr * 48 + k * 16, 16)]
                plsc.store_scatter(
                    rows_v.at[p], [rowoff[k] + gr * 16, coloff[k]], vals
                )
            return carry

        lax.fori_loop(0, GROUPS, body, 0)

    load_chunk(0, 0)
    for g in range(NCHUNK):
        p = g % 2
        for d in gather_descs(g, p):
            d.wait()
        if g + 1 < NCHUNK:
            if g >= 1:
                out_desc(g - 1, 1 - p).wait()
            load_chunk(g + 1, 1 - p)
        merge(p)
        out_desc(g, p).start()
    out_desc(NCHUNK - 1, (NCHUNK - 1) % 2).wait()


def kernel(pos, h, adj, embedding):
    emb0 = embedding.at[0].set(0.0)
    tbl = jnp.concatenate(
        [jnp.zeros((emb0.shape[0], 3), jnp.float32), emb0], axis=1
    )
    out = _sc_encode(tbl, h.reshape(-1), pos.reshape(-1))
    return out.reshape(B, N, D)


# speedup vs baseline: 9.7679x; 2.6485x over previous
import functools
import jax
import jax.numpy as jnp
from jax import lax
from jax.experimental import pallas as pl
from jax.experimental.pallas import tpu as pltpu
from jax.experimental.pallas import tpu_sc as plsc

B, N = 1024, 128
D = 128
ROWS = B * N
NW = 32
ROWS_PER_W = ROWS // NW
C = 256
BPC = C // N
NCHUNK = ROWS_PER_W // C
CPG = 8 // BPC
NGROUP = NCHUNK // CPG
GROUPS = C // 16

_mesh = plsc.VectorSubcoreMesh(core_axis_name="c", subcore_axis_name="s")


@functools.partial(
    pl.kernel,
    out_type=jax.ShapeDtypeStruct((ROWS, D), jnp.float32),
    mesh=_mesh,
    scratch_types=[
        pltpu.VMEM((2, 8, 128), jnp.int32),
        pltpu.VMEM((2, C, D), jnp.float32),
        pltpu.VMEM((2, 8, 3 * N), jnp.float32),
        pltpu.VMEM_SHARED((119, 128), jnp.float32),
        pltpu.SemaphoreType.DMA,
        pltpu.SemaphoreType.DMA,
        pltpu.SemaphoreType.DMA,
        pltpu.SemaphoreType.DMA,
    ],
    compiler_params=pltpu.CompilerParams(needs_layout_passes=False),
)
def _sc_encode(
    tbl_hbm, h_hbm, pos_hbm, out_hbm, idx_v, rows_v, pos_v, tbl_sh, g0, g1, o0, o1
):
    wid = lax.axis_index("s") * 2 + lax.axis_index("c")
    gsem = (g0, g1)
    osem = (o0, o1)

    @pl.when(lax.axis_index("s") == 0)
    def _stage():
        pltpu.sync_copy(tbl_hbm, tbl_sh)

    plsc.subcore_barrier()

    lane = jax.lax.broadcasted_iota(jnp.int32, (16,), 0)
    rowoff = []
    coloff = []
    for k in range(3):
        t = lane + (16 * k)
        rowoff.append(t // 3)
        coloff.append(t - 3 * (t // 3))

    def gather_descs(g, p):
        q = (g // CPG) % 2
        boff = (g % CPG) * BPC
        return [
            pltpu.make_async_copy(
                tbl_sh.at[idx_v.at[q, boff + j]],
                rows_v.at[p, pl.ds(j * 128, 128)],
                gsem[p],
            )
            for j in range(BPC)
        ]

    def out_desc(g, p):
        base = wid * ROWS_PER_W + g * C
        return pltpu.make_async_copy(
            rows_v.at[p], out_hbm.at[pl.ds(base, C)], osem[p]
        )

    def load_chunk(g, p):
        if g % CPG == 0:
            q = (g // CPG) % 2
            brow = wid * (ROWS_PER_W // N) + (g // CPG) * 8
            pltpu.sync_copy(h_hbm.at[pl.ds(brow, 8)], idx_v.at[q])
            pltpu.sync_copy(pos_hbm.at[pl.ds(brow, 8)], pos_v.at[q])
        for d in gather_descs(g, p):
            d.start()

    def merge(g, p):
        q = (g // CPG) % 2
        boff = (g % CPG) * BPC

        def body(gr, carry):
            lb = boff + gr // 8
            off = (gr % 8) * 48
            for k in range(3):
                vals = pos_v[q, lb, pl.ds(off + k * 16, 16)]
                plsc.store_scatter(
                    rows_v.at[p], [rowoff[k] + gr * 16, coloff[k]], vals
                )
            return carry

        lax.fori_loop(0, GROUPS, body, 0)

    load_chunk(0, 0)
    for g in range(NCHUNK):
        p = g % 2
        for d in gather_descs(g, p):
            d.wait()
        if g + 1 < NCHUNK:
            if g >= 1:
                out_desc(g - 1, 1 - p).wait()
            load_chunk(g + 1, 1 - p)
        merge(g, p)
        out_desc(g, p).start()
    out_desc(NCHUNK - 1, (NCHUNK - 1) % 2).wait()


def kernel(pos, h, adj, embedding):
    emb0 = embedding.at[0].set(0.0)
    tbl = jnp.concatenate(
        [jnp.zeros((emb0.shape[0], 3), jnp.float32), emb0], axis=1
    )
    out = _sc_encode(tbl, h, pos.reshape(B, 3 * N))
    return out.reshape(B, N, D)
